# baseline (speedup 1.0000x reference)
import functools

import jax
import jax.numpy as jnp
import numpy as np
from jax import lax
from jax.experimental import pallas as pl
from jax.experimental.pallas import tpu as pltpu
from jax.experimental.pallas import tpu_sc as plsc

B, N_E, D_R = 32, 128, 64
N_TRI = N_E * (N_E + 1) // 2
CH = 192
N_CH = N_TRI // CH
OUT_ROWS = N_TRI * D_R // 128
S1_ROWS = 280
MAX_P = 24


def _chunk_pieces(jc):
    lo, hi = jc * CH, (jc + 1) * CH
    pieces = []
    pbase = 0
    for r in range(N_E):
        off = r * (r + 1) // 2
        a, b_ = max(off, lo), min(off + r + 1, hi)
        if a < b_:
            clo8 = (a - off) & ~7
            cnt8 = min(-(-(b_ - off) // 8) * 8, N_E) - clo8
            src0 = pbase + (a - off) - clo8
            pieces.append((r, clo8, cnt8, pbase, a - lo, src0, b_ - a))
            pbase += cnt8
    assert pbase <= S1_ROWS, (jc, pbase)
    assert len(pieces) <= MAX_P, (jc, len(pieces))
    return pieces


def _desc_table() -> np.ndarray:
    tab = np.zeros((N_CH, MAX_P, 128), dtype=np.int32)
    for jc in range(N_CH):
        for p, (_, _, _, _, e0, src0, count) in enumerate(_chunk_pieces(jc)):
            tab[jc, p, 0] = e0
            tab[jc, p, 1] = src0
            tab[jc, p, 2] = count
    return tab


_TAB = _desc_table()


def _flatten_tri_sc(x, tab):
    mesh = plsc.VectorSubcoreMesh(core_axis_name="c", subcore_axis_name="s")

    @functools.partial(
        pl.kernel,
        mesh=mesh,
        compiler_params=pltpu.CompilerParams(use_tc_tiling_on_sc=True),
        out_type=jax.ShapeDtypeStruct((B, OUT_ROWS, 128), jnp.float32),
        scratch_types=[
            pltpu.VMEM((2, S1_ROWS, D_R), jnp.float32),
            pltpu.VMEM((2, CH // 2, 128), jnp.float32),
            pltpu.VMEM((2, MAX_P, 128), jnp.int32),
            pltpu.SemaphoreType.DMA,
            pltpu.SemaphoreType.DMA,
            pltpu.SemaphoreType.DMA,
            pltpu.SemaphoreType.DMA,
        ],
    )
    def k(in_hbm, tab_hbm, out_hbm, s1, s2, descv, f0, f1, d0, d1):
        wid = lax.axis_index("s") * 2 + lax.axis_index("c")
        fsem = (f0, f1)
        dsem = (d0, d1)

        def fill(jc):
            buf = jc % 2
            cps = [
                pltpu.async_copy(tab_hbm.at[jc], descv.at[buf], fsem[buf])
            ]
            for (r, clo8, cnt8, pbase, _, _, _) in _chunk_pieces(jc):
                cps.append(
                    pltpu.async_copy(
                        in_hbm.at[wid, r, pl.ds(clo8, cnt8)],
                        s1.at[buf, pl.ds(pbase, cnt8)],
                        fsem[buf],
                    )
                )
            return cps

        def bridge(buf):
            def piece_body(p, carry):
                v = descv[buf, p, pl.ds(0, 16)]
                e0, src0, count = v[0], v[1], v[2]
                i0 = lax.bitwise_and(e0, 1)

                @pl.when((i0 == 1) & (count > 0))
                def _head():
                    for k4 in range(4):
                        s2[buf, e0 // 2, pl.ds(64 + 16 * k4, 16)] = s1[
                            buf, src0, pl.ds(16 * k4, 16)
                        ]

                npairs = lax.max(count - i0, 0) // 2
                row0 = (e0 + i0) // 2
                s0 = src0 + i0

                def pair_body(t, c2):
                    for k4 in range(4):
                        s2[buf, row0 + t, pl.ds(16 * k4, 16)] = s1[
                            buf, s0 + 2 * t, pl.ds(16 * k4, 16)
                        ]
                        s2[buf, row0 + t, pl.ds(64 + 16 * k4, 16)] = s1[
                            buf, s0 + 2 * t + 1, pl.ds(16 * k4, 16)
                        ]
                    return c2

                lax.fori_loop(0, npairs, pair_body, 0)

                @pl.when(lax.bitwise_and(count - i0, 1) == 1)
                def _tail():
                    j = i0 + 2 * npairs
                    for k4 in range(4):
                        s2[buf, (e0 + j) // 2, pl.ds(16 * k4, 16)] = s1[
                            buf, src0 + j, pl.ds(16 * k4, 16)
                        ]

                return carry

            lax.fori_loop(0, MAX_P, piece_body, 0)

        def drain(jc):
            buf = jc % 2
            return pltpu.async_copy(
                s2.at[buf],
                out_hbm.at[wid, pl.ds(jc * (CH // 2), CH // 2)],
                dsem[buf],
            )

        pending_fills = {0: fill(0), 1: fill(1)}
        pending_drains = {}
        for jc in range(N_CH):
            for c in pending_fills.pop(jc):
                c.wait()
            if jc - 2 in pending_drains:
                pending_drains.pop(jc - 2).wait()
            bridge(jc % 2)
            pending_drains[jc] = drain(jc)
            if jc + 2 < N_CH:
                pending_fills[jc + 2] = fill(jc + 2)
        for jc in sorted(pending_drains):
            pending_drains.pop(jc).wait()

    return k(x, tab)


def kernel(inputs):
    tab = jnp.asarray(_TAB)
    out = _flatten_tri_sc(inputs, tab)
    return out.reshape(B, N_TRI * D_R)


# speedup vs baseline: 1.0974x; 1.0974x over previous
import functools

import jax
import jax.numpy as jnp
import numpy as np
from jax import lax
from jax.experimental import pallas as pl
from jax.experimental.pallas import tpu as pltpu
from jax.experimental.pallas import tpu_sc as plsc

B, N_E, D_R = 32, 128, 64
N_TRI = N_E * (N_E + 1) // 2
CH = 192
N_CH = N_TRI // CH
OUT_ROWS = N_TRI * D_R // 128
S1_ROWS = 280
MAX_P = 24


def _chunk_pieces(jc):
    lo, hi = jc * CH, (jc + 1) * CH
    pieces = []
    pbase = 0
    for r in range(N_E):
        off = r * (r + 1) // 2
        a, b_ = max(off, lo), min(off + r + 1, hi)
        if a < b_:
            clo8 = (a - off) & ~7
            cnt8 = min(-(-(b_ - off) // 8) * 8, N_E) - clo8
            src0 = pbase + (a - off) - clo8
            pieces.append((r, clo8, cnt8, pbase, a - lo, src0, b_ - a))
            pbase += cnt8
    assert pbase <= S1_ROWS, (jc, pbase)
    assert len(pieces) <= MAX_P, (jc, len(pieces))
    return pieces


def _desc_table() -> np.ndarray:
    tab = np.zeros((N_CH, MAX_P, 128), dtype=np.int32)
    for jc in range(N_CH):
        for p, (_, _, _, _, e0, src0, count) in enumerate(_chunk_pieces(jc)):
            tab[jc, p, 0] = e0
            tab[jc, p, 1] = src0
            tab[jc, p, 2] = count
    return tab


_TAB = _desc_table()


def _flatten_tri_sc(x, tab):
    mesh = plsc.VectorSubcoreMesh(core_axis_name="c", subcore_axis_name="s")

    @functools.partial(
        pl.kernel,
        mesh=mesh,
        compiler_params=pltpu.CompilerParams(use_tc_tiling_on_sc=True),
        out_type=jax.ShapeDtypeStruct((B, OUT_ROWS, 128), jnp.float32),
        scratch_types=[
            pltpu.VMEM((2, S1_ROWS, D_R), jnp.float32),
            pltpu.VMEM((2, CH // 2, 128), jnp.float32),
            pltpu.VMEM((2, MAX_P, 128), jnp.int32),
            pltpu.SemaphoreType.DMA,
            pltpu.SemaphoreType.DMA,
            pltpu.SemaphoreType.DMA,
            pltpu.SemaphoreType.DMA,
        ],
    )
    def k(in_hbm, tab_hbm, out_hbm, s1, s2, descv, f0, f1, d0, d1):
        wid = lax.axis_index("s") * 2 + lax.axis_index("c")
        fsem = (f0, f1)
        dsem = (d0, d1)

        def fill(jc):
            buf = jc % 2
            cps = [
                pltpu.async_copy(tab_hbm.at[jc], descv.at[buf], fsem[buf])
            ]
            for (r, clo8, cnt8, pbase, _, _, _) in _chunk_pieces(jc):
                cps.append(
                    pltpu.async_copy(
                        in_hbm.at[wid, r, pl.ds(clo8, cnt8)],
                        s1.at[buf, pl.ds(pbase, cnt8)],
                        fsem[buf],
                    )
                )
            return cps

        def bridge(buf, npieces):
            def piece_body(p, carry):
                v = descv[buf, p, pl.ds(0, 16)]
                e0, src0, count = v[0], v[1], v[2]
                i0 = lax.bitwise_and(e0, 1)

                @pl.when((i0 == 1) & (count > 0))
                def _head():
                    for k4 in range(4):
                        s2[buf, e0 // 2, pl.ds(64 + 16 * k4, 16)] = s1[
                            buf, src0, pl.ds(16 * k4, 16)
                        ]

                npairs = lax.max(count - i0, 0) // 2
                row0 = (e0 + i0) // 2
                s0 = src0 + i0

                def pair_body(t, c2):
                    for k4 in range(4):
                        s2[buf, row0 + t, pl.ds(16 * k4, 16)] = s1[
                            buf, s0 + 2 * t, pl.ds(16 * k4, 16)
                        ]
                        s2[buf, row0 + t, pl.ds(64 + 16 * k4, 16)] = s1[
                            buf, s0 + 2 * t + 1, pl.ds(16 * k4, 16)
                        ]
                    return c2

                lax.fori_loop(0, npairs, pair_body, 0)

                @pl.when(lax.bitwise_and(count - i0, 1) == 1)
                def _tail():
                    j = i0 + 2 * npairs
                    for k4 in range(4):
                        s2[buf, (e0 + j) // 2, pl.ds(16 * k4, 16)] = s1[
                            buf, src0 + j, pl.ds(16 * k4, 16)
                        ]

                return carry

            lax.fori_loop(0, npieces, piece_body, 0)

        def drain(jc):
            buf = jc % 2
            return pltpu.async_copy(
                s2.at[buf],
                out_hbm.at[wid, pl.ds(jc * (CH // 2), CH // 2)],
                dsem[buf],
            )

        pending_fills = {0: fill(0), 1: fill(1)}
        pending_drains = {}
        for jc in range(N_CH):
            for c in pending_fills.pop(jc):
                c.wait()
            if jc - 2 in pending_drains:
                pending_drains.pop(jc - 2).wait()
            bridge(jc % 2, len(_chunk_pieces(jc)))
            pending_drains[jc] = drain(jc)
            if jc + 2 < N_CH:
                pending_fills[jc + 2] = fill(jc + 2)
        for jc in sorted(pending_drains):
            pending_drains.pop(jc).wait()

    return k(x, tab)


def kernel(inputs):
    tab = jnp.asarray(_TAB)
    out = _flatten_tri_sc(inputs, tab)
    return out.reshape(B, N_TRI * D_R)
